# Initial kernel scaffold; baseline (speedup 1.0000x reference)
import functools

import jax
import jax.numpy as jnp
from jax import lax
from jax.experimental import pallas as pl
from jax.experimental.pallas import tpu as pltpu
from jax.experimental.pallas import tpu_sc as plsc

N = 10000
E = 320000
NPAD = 10240
IN_CH = 128
HID = 128
OUT = 64

NC = 2
NS = 16
NW = NC * NS
K = 128
ITERS = -(-E // (NW * K))
E_PAD = NW * K * ITERS
EPT = K * ITERS

_F32 = jnp.float32
_HIGHEST = lax.Precision.HIGHEST


def _dot(a, b):
    return lax.dot_general(a, b, (((1,), (0,)), ((), ())),
                           precision=_HIGHEST, preferred_element_type=_F32)


_MESH = plsc.VectorSubcoreMesh(core_axis_name="c", subcore_axis_name="s",
                               num_cores=NC, num_subcores=NS)


def _fill_rows(ref, rows, width, value):
    def body(i, _):
        for j in range(width // 16):
            ref[i, pl.ds(j * 16, 16)] = jnp.full((16,), value, _F32)
        return 0
    lax.fori_loop(0, rows, body, 0)


def _deg_body(dst_hbm, out_hbm, dst_v, ones_v, zero_v, acc_sh, sem):
    c = lax.axis_index("c")
    s = lax.axis_index("s")
    wid = c * NS + s
    _fill_rows(ones_v, K, 16, 1.0)
    _fill_rows(zero_v, K, 16, 0.0)
    rpt = NPAD // NS
    for jj in range(rpt // K):
        pltpu.sync_copy(zero_v, acc_sh.at[pl.ds(s * rpt + jj * K, K)])
    plsc.subcore_barrier()
    base = wid * EPT
    def body(k, _):
        off = pl.multiple_of(base + k * K, K)
        pltpu.sync_copy(dst_hbm.at[pl.ds(off, K)], dst_v)
        pltpu.sync_copy(ones_v, acc_sh.at[dst_v], add=True)
        return 0
    lax.fori_loop(0, ITERS, body, 0)
    plsc.subcore_barrier()
    for jj in range(rpt // K):
        r0 = s * rpt + jj * K
        pltpu.sync_copy(acc_sh.at[pl.ds(r0, K)], out_hbm.at[c, pl.ds(r0, K)])


_deg_kernel = pl.kernel(
    _deg_body,
    out_type=jax.ShapeDtypeStruct((NC, NPAD, 16), _F32),
    mesh=_MESH,
    scratch_types=[
        pltpu.VMEM((K,), jnp.int32),
        pltpu.VMEM((K, 16), _F32),
        pltpu.VMEM((K, 16), _F32),
        pltpu.VMEM_SHARED((NPAD, 16), _F32),
        pltpu.SemaphoreType.DMA,
    ],
)


def _make_spmm(D):
    def body(hs_hbm, src_hbm, dst_hbm, out_hbm, src_v, dst_v, rows_v, acc_sh,
             sem):
        c = lax.axis_index("c")
        s = lax.axis_index("s")
        wid = c * NS + s
        _fill_rows(rows_v, K, D, 0.0)
        rpt = NPAD // NS
        for jj in range(rpt // K):
            pltpu.sync_copy(rows_v, acc_sh.at[pl.ds(s * rpt + jj * K, K)])
        plsc.subcore_barrier()
        base = wid * EPT
        def step(k, _):
            off = pl.multiple_of(base + k * K, K)
            pltpu.sync_copy(src_hbm.at[pl.ds(off, K)], src_v)
            pltpu.sync_copy(dst_hbm.at[pl.ds(off, K)], dst_v)
            pltpu.async_copy(hs_hbm.at[src_v], rows_v, sem).wait()
            pltpu.sync_copy(rows_v, acc_sh.at[dst_v], add=True)
            return 0
        lax.fori_loop(0, ITERS, step, 0)
        plsc.subcore_barrier()
        for jj in range(rpt // K):
            r0 = s * rpt + jj * K
            pltpu.sync_copy(acc_sh.at[pl.ds(r0, K)], out_hbm.at[c, pl.ds(r0, K)])

    return pl.kernel(
        body,
        out_type=jax.ShapeDtypeStruct((NC, NPAD, D), _F32),
        mesh=_MESH,
        scratch_types=[
            pltpu.VMEM((K,), jnp.int32),
            pltpu.VMEM((K,), jnp.int32),
            pltpu.VMEM((K, D), _F32),
            pltpu.VMEM_SHARED((NPAD, D), _F32),
            pltpu.SemaphoreType.DMA,
        ],
    )


_spmm_hid = _make_spmm(HID)
_spmm_out = _make_spmm(OUT)


_BLK = 1024
_GRID = NPAD // _BLK


def _dinv_from(degp_ref):
    deg = degp_ref[0, :, 0:1] + degp_ref[1, :, 0:1] + 1.0
    return lax.rsqrt(deg)


def _tcA_body(x_ref, w1_ref, degp_ref, hs_ref):
    dinv = _dinv_from(degp_ref)
    hs_ref[...] = _dot(x_ref[...], w1_ref[...]) * dinv


def _tcB_body(p_ref, hs_ref, degp_ref, w2_ref, b1_ref, h2s_ref):
    dinv = _dinv_from(degp_ref)
    z = (p_ref[0] + p_ref[1] + hs_ref[...]) * dinv + b1_ref[...]
    z = jnp.maximum(z, 0.0)
    h2s_ref[...] = _dot(z, w2_ref[...]) * dinv


def _tcC_body(p_ref, h2s_ref, degp_ref, b2_ref, out_ref):
    dinv = _dinv_from(degp_ref)
    logits = (p_ref[0] + p_ref[1] + h2s_ref[...]) * dinv + b2_ref[...]
    m = jnp.max(logits, axis=1, keepdims=True)
    e = jnp.exp(logits - m)
    out_ref[...] = e / jnp.sum(e, axis=1, keepdims=True)


def _row_spec(d):
    return pl.BlockSpec((_BLK, d), lambda i: (i, 0))


def _pair_spec(d):
    return pl.BlockSpec((NC, _BLK, d), lambda i: (0, i, 0))


def _full_spec(r, d):
    return pl.BlockSpec((r, d), lambda i: (0, 0))


_tcA = pl.pallas_call(
    _tcA_body,
    grid=(_GRID,),
    in_specs=[_row_spec(IN_CH), _full_spec(IN_CH, HID), _pair_spec(16)],
    out_specs=_row_spec(HID),
    out_shape=jax.ShapeDtypeStruct((NPAD, HID), _F32),
)

_tcB = pl.pallas_call(
    _tcB_body,
    grid=(_GRID,),
    in_specs=[_pair_spec(HID), _row_spec(HID), _pair_spec(16),
              _full_spec(HID, OUT), _full_spec(1, HID)],
    out_specs=_row_spec(OUT),
    out_shape=jax.ShapeDtypeStruct((NPAD, OUT), _F32),
)

_tcC = pl.pallas_call(
    _tcC_body,
    grid=(_GRID,),
    in_specs=[_pair_spec(OUT), _row_spec(OUT), _pair_spec(16),
              _full_spec(1, OUT)],
    out_specs=_row_spec(OUT),
    out_shape=jax.ShapeDtypeStruct((NPAD, OUT), _F32),
)


@jax.jit
def kernel(x, edge_index, W1, b1, W2, b2):
    src = edge_index[0].astype(jnp.int32)
    dst = edge_index[1].astype(jnp.int32)
    pad_e = E_PAD - E
    src_p = jnp.concatenate([src, jnp.full((pad_e,), N, jnp.int32)])
    dst_p = jnp.concatenate([dst, jnp.full((pad_e,), N, jnp.int32)])
    x_p = jnp.pad(x, ((0, NPAD - N), (0, 0)))

    degp = _deg_kernel(dst_p)
    hs = _tcA(x_p, W1, degp)
    p1 = _spmm_hid(hs, src_p, dst_p)
    h2s = _tcB(p1, hs, degp, W2, b1.reshape(1, HID))
    p2 = _spmm_out(h2s, src_p, dst_p)
    out = _tcC(p2, h2s, degp, b2.reshape(1, OUT))
    return out[:N]


# speedup vs baseline: 12.4777x; 12.4777x over previous
import functools

import jax
import jax.numpy as jnp
from jax import lax
from jax.experimental import pallas as pl
from jax.experimental.pallas import tpu as pltpu
from jax.experimental.pallas import tpu_sc as plsc

N = 10000
E = 320000
NPAD = 10240
IN_CH = 128
HID = 128
OUT = 64

NC = 2
NS = 16
NW = NC * NS
K = 128
ITERS = -(-E // (NW * K))
E_PAD = NW * K * ITERS
EPT = K * ITERS

_F32 = jnp.float32
_HIGHEST = lax.Precision.HIGHEST


def _dot(a, b):
    return lax.dot_general(a, b, (((1,), (0,)), ((), ())),
                           precision=_HIGHEST, preferred_element_type=_F32)


_MESH = plsc.VectorSubcoreMesh(core_axis_name="c", subcore_axis_name="s",
                               num_cores=NC, num_subcores=NS)


def _fill_rows(ref, rows, width, value):
    def body(i, _):
        for j in range(width // 16):
            ref[i, pl.ds(j * 16, 16)] = jnp.full((16,), value, _F32)
        return 0
    lax.fori_loop(0, rows, body, 0)


def _deg_body(dst_hbm, out_hbm, dst_v, ones_v, zero_v, acc_sh, sem):
    c = lax.axis_index("c")
    s = lax.axis_index("s")
    wid = c * NS + s
    _fill_rows(ones_v, K, 16, 1.0)
    _fill_rows(zero_v, K, 16, 0.0)
    rpt = NPAD // NS
    for jj in range(rpt // K):
        pltpu.sync_copy(zero_v, acc_sh.at[pl.ds(s * rpt + jj * K, K)])
    plsc.subcore_barrier()
    base = wid * EPT
    def body(k, _):
        off = pl.multiple_of(base + k * K, K)
        pltpu.sync_copy(dst_hbm.at[pl.ds(off, K)], dst_v)
        pltpu.sync_copy(ones_v, acc_sh.at[dst_v], add=True)
        return 0
    lax.fori_loop(0, ITERS, body, 0)
    plsc.subcore_barrier()
    for jj in range(rpt // K):
        r0 = s * rpt + jj * K
        pltpu.sync_copy(acc_sh.at[pl.ds(r0, K)], out_hbm.at[c, pl.ds(r0, K)])


_deg_kernel = pl.kernel(
    _deg_body,
    out_type=jax.ShapeDtypeStruct((NC, NPAD, 16), _F32),
    mesh=_MESH,
    scratch_types=[
        pltpu.VMEM((K,), jnp.int32),
        pltpu.VMEM((K, 16), _F32),
        pltpu.VMEM((K, 16), _F32),
        pltpu.VMEM_SHARED((NPAD, 16), _F32),
        pltpu.SemaphoreType.DMA,
    ],
)


def _make_spmm(D):
    def body(hs_hbm, src_hbm, dst_hbm, out_hbm, src_v, dst_v, rows_v, acc_sh,
             sem):
        c = lax.axis_index("c")
        s = lax.axis_index("s")
        wid = c * NS + s
        _fill_rows(rows_v, K, D, 0.0)
        rpt = NPAD // NS
        for jj in range(rpt // K):
            pltpu.sync_copy(rows_v, acc_sh.at[pl.ds(s * rpt + jj * K, K)])
        plsc.subcore_barrier()
        base = wid * EPT
        def step(k, _):
            off = pl.multiple_of(base + k * K, K)
            pltpu.sync_copy(src_hbm.at[pl.ds(off, K)], src_v)
            pltpu.sync_copy(dst_hbm.at[pl.ds(off, K)], dst_v)
            pltpu.async_copy(hs_hbm.at[src_v], rows_v, sem).wait()
            pltpu.sync_copy(rows_v, acc_sh.at[dst_v], add=True)
            return 0
        lax.fori_loop(0, ITERS, step, 0)
        plsc.subcore_barrier()
        for jj in range(rpt // K):
            r0 = s * rpt + jj * K
            pltpu.sync_copy(acc_sh.at[pl.ds(r0, K)], out_hbm.at[c, pl.ds(r0, K)])

    return pl.kernel(
        body,
        out_type=jax.ShapeDtypeStruct((NC, NPAD, D), _F32),
        mesh=_MESH,
        compiler_params=pltpu.CompilerParams(use_tc_tiling_on_sc=False),
        scratch_types=[
            pltpu.VMEM((K,), jnp.int32),
            pltpu.VMEM((K,), jnp.int32),
            pltpu.VMEM((K, D), _F32),
            pltpu.VMEM_SHARED((NPAD, D), _F32),
            pltpu.SemaphoreType.DMA,
        ],
    )


_spmm_hid = _make_spmm(HID)
_spmm_out = _make_spmm(OUT)


_BLK = 1024
_GRID = NPAD // _BLK


def _dinv_from(degp_ref):
    deg = degp_ref[0, :, 0:1] + degp_ref[1, :, 0:1] + 1.0
    return lax.rsqrt(deg)


def _tcA_body(x_ref, w1_ref, degp_ref, hs_ref):
    dinv = _dinv_from(degp_ref)
    hs_ref[...] = _dot(x_ref[...], w1_ref[...]) * dinv


def _tcB_body(p_ref, hs_ref, degp_ref, w2_ref, b1_ref, h2s_ref):
    dinv = _dinv_from(degp_ref)
    z = (p_ref[0] + p_ref[1] + hs_ref[...]) * dinv + b1_ref[...]
    z = jnp.maximum(z, 0.0)
    h2s_ref[...] = _dot(z, w2_ref[...]) * dinv


def _tcC_body(p_ref, h2s_ref, degp_ref, b2_ref, out_ref):
    dinv = _dinv_from(degp_ref)
    logits = (p_ref[0] + p_ref[1] + h2s_ref[...]) * dinv + b2_ref[...]
    m = jnp.max(logits, axis=1, keepdims=True)
    e = jnp.exp(logits - m)
    out_ref[...] = e / jnp.sum(e, axis=1, keepdims=True)


def _row_spec(d):
    return pl.BlockSpec((_BLK, d), lambda i: (i, 0))


def _pair_spec(d):
    return pl.BlockSpec((NC, _BLK, d), lambda i: (0, i, 0))


def _full_spec(r, d):
    return pl.BlockSpec((r, d), lambda i: (0, 0))


_tcA = pl.pallas_call(
    _tcA_body,
    grid=(_GRID,),
    in_specs=[_row_spec(IN_CH), _full_spec(IN_CH, HID), _pair_spec(16)],
    out_specs=_row_spec(HID),
    out_shape=jax.ShapeDtypeStruct((NPAD, HID), _F32),
)

_tcB = pl.pallas_call(
    _tcB_body,
    grid=(_GRID,),
    in_specs=[_pair_spec(HID), _row_spec(HID), _pair_spec(16),
              _full_spec(HID, OUT), _full_spec(1, HID)],
    out_specs=_row_spec(OUT),
    out_shape=jax.ShapeDtypeStruct((NPAD, OUT), _F32),
)

_tcC = pl.pallas_call(
    _tcC_body,
    grid=(_GRID,),
    in_specs=[_pair_spec(OUT), _row_spec(OUT), _pair_spec(16),
              _full_spec(1, OUT)],
    out_specs=_row_spec(OUT),
    out_shape=jax.ShapeDtypeStruct((NPAD, OUT), _F32),
)


@jax.jit
def kernel(x, edge_index, W1, b1, W2, b2):
    src = edge_index[0].astype(jnp.int32)
    dst = edge_index[1].astype(jnp.int32)
    pad_e = E_PAD - E
    src_p = jnp.concatenate([src, jnp.full((pad_e,), N, jnp.int32)])
    dst_p = jnp.concatenate([dst, jnp.full((pad_e,), N, jnp.int32)])
    x_p = jnp.pad(x, ((0, NPAD - N), (0, 0)))

    degp = _deg_kernel(dst_p)
    hs = _tcA(x_p, W1, degp)
    p1 = _spmm_hid(hs, src_p, dst_p)
    h2s = _tcB(p1, hs, degp, W2, b1.reshape(1, HID))
    p2 = _spmm_out(h2s, src_p, dst_p)
    out = _tcC(p2, h2s, degp, b2.reshape(1, OUT))
    return out[:N]
